# baseline (speedup 1.0000x reference)
import jax
import jax.numpy as jnp
from jax import lax
from jax.experimental import pallas as pl
from jax.experimental.pallas import tpu as pltpu
from jax.experimental.pallas import tpu_sc as plsc

N_TOK = 1024
D = 128
NC = 2
NS = 16
NW = NC * NS
IPW = N_TOK // NW
TAB_ROWS = 2 * N_TOK


def _sc_body(table_hbm, out_hbm, shared, sem_out):
    c = lax.axis_index("c")
    s = lax.axis_index("s")
    wid = s * NC + c
    i0 = wid * IPW

    @pl.when(s == 0)
    def _():
        pltpu.sync_copy(table_hbm, shared)

    plsc.subcore_barrier()
    copies = []
    for ii in range(IPW):
        i = i0 + ii
        cp = pltpu.make_async_copy(
            shared.at[pl.ds((N_TOK - 1 - i) * D, N_TOK * D)],
            out_hbm.at[pl.ds(i * N_TOK * D, N_TOK * D)],
            sem_out,
        )
        cp.start()
        copies.append(cp)
    for cp in copies:
        cp.wait()


def kernel(x, emb_table):
    del x
    table_flat = jnp.concatenate(
        [emb_table, jnp.zeros((1, D), jnp.float32)], axis=0
    ).reshape(-1)
    flat = pl.kernel(
        _sc_body,
        out_type=jax.ShapeDtypeStruct((N_TOK * N_TOK * D,), jnp.float32),
        scratch_types=[
            pltpu.VMEM_SHARED((TAB_ROWS * D,), jnp.float32),
            pltpu.SemaphoreType.DMA,
        ],
        mesh=plsc.VectorSubcoreMesh(core_axis_name="c", subcore_axis_name="s"),
    )(table_flat)
    return flat.reshape(N_TOK, N_TOK, D)


# speedup vs baseline: 1.4697x; 1.4697x over previous
import jax
import jax.numpy as jnp
from jax import lax
from jax.experimental import pallas as pl
from jax.experimental.pallas import tpu as pltpu
from jax.experimental.pallas import tpu_sc as plsc

N_TOK = 1024
D = 128
NC = 2
NS = 16
NW = NC * NS
IPW = N_TOK // NW
SPLIT = 20
JCH = 512
NJC = N_TOK // JCH
BUF_ROWS = 536
TAB_ROWS = 2 * N_TOK + 8


def _sc_body(table_hbm, out_hbm, buf, shared, sem_out, sem_spm):
    c = lax.axis_index("c")
    s = lax.axis_index("s")
    wid = s * NC + c
    i0 = wid * IPW

    @pl.when(s == 0)
    def _():
        pltpu.sync_copy(table_hbm, shared)

    plsc.subcore_barrier()

    spm_cps = []
    for ii in range(SPLIT, IPW):
        i = i0 + ii
        cp = pltpu.make_async_copy(
            shared.at[pl.ds((N_TOK - 1 - i) * D, N_TOK * D)],
            out_hbm.at[pl.ds(i * N_TOK * D, N_TOK * D)],
            sem_spm,
        )
        cp.start()
        spm_cps.append(cp)

    for jc in range(NJC):
        j0 = jc * JCH
        s0 = (N_TOK - 1) + j0 - i0 - (SPLIT - 1)
        pltpu.sync_copy(table_hbm.at[pl.ds(s0 * D, BUF_ROWS * D)], buf)
        copies = []
        for ii in range(SPLIT):
            i = i0 + ii
            cp = pltpu.make_async_copy(
                buf.at[pl.ds((SPLIT - 1 - ii) * D, JCH * D)],
                out_hbm.at[pl.ds((i * N_TOK + j0) * D, JCH * D)],
                sem_out,
            )
            cp.start()
            copies.append(cp)
        for cp in copies:
            cp.wait()

    for cp in spm_cps:
        cp.wait()


def kernel(x, emb_table):
    del x
    table_flat = jnp.concatenate(
        [emb_table, jnp.zeros((TAB_ROWS - (2 * N_TOK - 1), D), jnp.float32)],
        axis=0,
    ).reshape(-1)
    flat = pl.kernel(
        _sc_body,
        out_type=jax.ShapeDtypeStruct((N_TOK * N_TOK * D,), jnp.float32),
        scratch_types=[
            pltpu.VMEM((BUF_ROWS * D,), jnp.float32),
            pltpu.VMEM_SHARED((TAB_ROWS * D,), jnp.float32),
            pltpu.SemaphoreType.DMA,
            pltpu.SemaphoreType.DMA,
        ],
        mesh=plsc.VectorSubcoreMesh(core_axis_name="c", subcore_axis_name="s"),
    )(table_flat)
    return flat.reshape(N_TOK, N_TOK, D)


# speedup vs baseline: 1.4717x; 1.0014x over previous
import jax
import jax.numpy as jnp
from jax import lax
from jax.experimental import pallas as pl
from jax.experimental.pallas import tpu as pltpu
from jax.experimental.pallas import tpu_sc as plsc

N_TOK = 1024
D = 128
NC = 2
NS = 16
NW = NC * NS
IPW = N_TOK // NW
JCH = 512
NJC = N_TOK // JCH
BUF_ROWS = JCH + IPW


def _sc_body(table_hbm, out_hbm, buf, sem_out):
    c = lax.axis_index("c")
    s = lax.axis_index("s")
    wid = s * NC + c
    i0 = wid * IPW
    for jc in range(NJC):
        j0 = jc * JCH
        s0 = (N_TOK - 1) + j0 - i0 - (IPW - 1)
        pltpu.sync_copy(table_hbm.at[pl.ds(s0 * D, BUF_ROWS * D)], buf)
        copies = []
        for ii in range(IPW):
            i = i0 + ii
            cp = pltpu.make_async_copy(
                buf.at[pl.ds((IPW - 1 - ii) * D, JCH * D)],
                out_hbm.at[pl.ds((i * N_TOK + j0) * D, JCH * D)],
                sem_out,
            )
            cp.start()
            copies.append(cp)
        for cp in copies:
            cp.wait()


def kernel(x, emb_table):
    del x
    table_flat = jnp.concatenate(
        [emb_table, jnp.zeros((1, D), jnp.float32)], axis=0
    ).reshape(-1)
    flat = pl.kernel(
        _sc_body,
        out_type=jax.ShapeDtypeStruct((N_TOK * N_TOK * D,), jnp.float32),
        scratch_types=[
            pltpu.VMEM((BUF_ROWS * D,), jnp.float32),
            pltpu.SemaphoreType.DMA,
        ],
        mesh=plsc.VectorSubcoreMesh(core_axis_name="c", subcore_axis_name="s"),
    )(table_flat)
    return flat.reshape(N_TOK, N_TOK, D)
